# baseline (speedup 1.0000x reference)
import functools

import jax
import jax.numpy as jnp
from jax import lax
from jax.experimental import pallas as pl
from jax.experimental.pallas import tpu as pltpu
from jax.experimental.pallas import tpu_sc as plsc

EMBD_DIM = 128
EPS = 1e-05
NC = 2
NS = 16
NW = NC * NS
LANES = 16
KV = EMBD_DIM // LANES


def _emb_ln(x, embd, pos_embd, gamma, beta, *, batch, seq_len):
    spw = seq_len // NW
    mesh = plsc.VectorSubcoreMesh(core_axis_name="c", subcore_axis_name="s")

    @functools.partial(
        pl.kernel,
        mesh=mesh,
        compiler_params=pltpu.CompilerParams(needs_layout_passes=False),
        out_type=jax.ShapeDtypeStruct((batch, seq_len, EMBD_DIM), jnp.float32),
        scratch_types=[
            pltpu.VMEM((batch, spw), jnp.int32),
            pltpu.VMEM((batch * spw, EMBD_DIM), jnp.float32),
            pltpu.VMEM((spw, EMBD_DIM), jnp.float32),
            pltpu.VMEM((EMBD_DIM,), jnp.float32),
            pltpu.VMEM((EMBD_DIM,), jnp.float32),
            pltpu.SemaphoreType.DMA,
            pltpu.SemaphoreType.DMA,
            pltpu.SemaphoreType.DMA,
            pltpu.SemaphoreType.DMA,
        ],
    )
    def k(x_hbm, embd_hbm, pos_hbm, gamma_hbm, beta_hbm, out_hbm,
          idx_v, rows_v, pos_v, g_v, b_v,
          isem, msem, osem, g0sem):
        wid = lax.axis_index("s") * NC + lax.axis_index("c")
        sbase = wid * spw
        gsems = [g0sem]
        nch = len(gsems)
        cpw = spw // nch

        idx_cp = [pltpu.async_copy(
            x_hbm.at[b, pl.ds(sbase, spw)], idx_v.at[b], isem)
            for b in range(batch)]
        misc = [pltpu.async_copy(pos_hbm.at[pl.ds(sbase, spw)], pos_v, msem),
                pltpu.async_copy(gamma_hbm, g_v, msem),
                pltpu.async_copy(beta_hbm, b_v, msem)]
        for c in idx_cp:
            c.wait()
        gath = [[pltpu.async_copy(
            embd_hbm.at[idx_v.at[b, pl.ds(c * cpw, cpw)]],
            rows_v.at[pl.ds(b * spw + c * cpw, cpw)], gsems[c])
            for b in range(batch)] for c in range(nch)]
        for c in misc:
            c.wait()

        gs = [g_v[pl.ds(t * LANES, LANES)] for t in range(KV)]
        bs = [b_v[pl.ds(t * LANES, LANES)] for t in range(KV)]
        inv_d = jnp.float32(1.0 / EMBD_DIM)

        def pos_body(i):
            ps = [pos_v[i, pl.ds(t * LANES, LANES)] for t in range(KV)]
            for b in range(batch):
                r = b * spw + i
                hs = [rows_v[r, pl.ds(t * LANES, LANES)] + ps[t]
                      for t in range(KV)]
                s = hs[0]
                for t in range(1, KV):
                    s = s + hs[t]
                q = hs[0] * hs[0]
                for t in range(1, KV):
                    q = q + hs[t] * hs[t]
                mean = jnp.sum(s) * inv_d
                ex2 = jnp.sum(q) * inv_d
                var = ex2 - mean * mean
                ve = var + EPS
                bits = lax.bitcast_convert_type(ve, jnp.int32)
                bits = jnp.int32(0x5F3759DF) - (bits >> 1)
                y = lax.bitcast_convert_type(bits, jnp.float32)
                half = jnp.float32(0.5) * ve
                for _ in range(2):
                    y = y * (jnp.float32(1.5) - half * y * y)
                yv = jnp.full((LANES,), y, dtype=jnp.float32)
                c0 = jnp.full((LANES,), -mean * y, dtype=jnp.float32)
                for t in range(KV):
                    rows_v[r, pl.ds(t * LANES, LANES)] = (
                        (hs[t] * yv + c0) * gs[t] + bs[t])

        outs = []
        for c in range(nch):
            for cp in gath[c]:
                cp.wait()
            pl.loop(c * cpw, (c + 1) * cpw, unroll=2)(pos_body)
            outs.extend(pltpu.async_copy(
                rows_v.at[pl.ds(b * spw + c * cpw, cpw)],
                out_hbm.at[b, pl.ds(sbase + c * cpw, cpw)], osem)
                for b in range(batch))
        for cp in outs:
            cp.wait()

    return k(x, embd, pos_embd, gamma, beta)


def kernel(x, embd, pos_embd, gamma, beta):
    b, s = x.shape
    return _emb_ln(x.astype(jnp.int32), embd, pos_embd, gamma, beta,
                   batch=b, seq_len=s)


# speedup vs baseline: 1.0058x; 1.0058x over previous
import functools

import jax
import jax.numpy as jnp
from jax import lax
from jax.experimental import pallas as pl
from jax.experimental.pallas import tpu as pltpu
from jax.experimental.pallas import tpu_sc as plsc

EMBD_DIM = 128
EPS = 1e-05
NC = 2
NS = 16
NW = NC * NS
LANES = 16
KV = EMBD_DIM // LANES


def _emb_ln(x, embd, pos_embd, gamma, beta, *, batch, seq_len):
    spw = seq_len // NW
    mesh = plsc.VectorSubcoreMesh(core_axis_name="c", subcore_axis_name="s")

    @functools.partial(
        pl.kernel,
        mesh=mesh,
        compiler_params=pltpu.CompilerParams(needs_layout_passes=False),
        out_type=jax.ShapeDtypeStruct((batch, seq_len, EMBD_DIM), jnp.float32),
        scratch_types=[
            pltpu.VMEM((batch, spw), jnp.int32),
            pltpu.VMEM((batch * spw, EMBD_DIM), jnp.float32),
            pltpu.VMEM((spw, EMBD_DIM), jnp.float32),
            pltpu.VMEM((EMBD_DIM,), jnp.float32),
            pltpu.VMEM((EMBD_DIM,), jnp.float32),
            pltpu.SemaphoreType.DMA,
            pltpu.SemaphoreType.DMA,
            pltpu.SemaphoreType.DMA,
            pltpu.SemaphoreType.DMA,
        ],
    )
    def k(x_hbm, embd_hbm, pos_hbm, gamma_hbm, beta_hbm, out_hbm,
          idx_v, rows_v, pos_v, g_v, b_v,
          isem, msem, osem, g0sem):
        wid = lax.axis_index("s") * NC + lax.axis_index("c")
        sbase = wid * spw
        gsems = [g0sem]
        nch = len(gsems)
        cpw = spw // nch

        idx_cp = [pltpu.async_copy(
            x_hbm.at[b, pl.ds(sbase, spw)], idx_v.at[b], isem)
            for b in range(batch)]
        misc = [pltpu.async_copy(pos_hbm.at[pl.ds(sbase, spw)], pos_v, msem),
                pltpu.async_copy(gamma_hbm, g_v, msem),
                pltpu.async_copy(beta_hbm, b_v, msem)]
        for c in idx_cp:
            c.wait()
        gath = [[pltpu.async_copy(
            embd_hbm.at[idx_v.at[b, pl.ds(c * cpw, cpw)]],
            rows_v.at[pl.ds(b * spw + c * cpw, cpw)], gsems[c])
            for b in range(batch)] for c in range(nch)]
        for c in misc:
            c.wait()

        gs = [g_v[pl.ds(t * LANES, LANES)] for t in range(KV)]
        bs = [b_v[pl.ds(t * LANES, LANES)] for t in range(KV)]
        inv_d = jnp.float32(1.0 / EMBD_DIM)

        def pos_body(i):
            ps = [pos_v[i, pl.ds(t * LANES, LANES)] for t in range(KV)]
            for b in range(batch):
                r = b * spw + i
                hs = [rows_v[r, pl.ds(t * LANES, LANES)] + ps[t]
                      for t in range(KV)]
                sl = hs
                while len(sl) > 1:
                    sl = [sl[j] + sl[j + 1] for j in range(0, len(sl), 2)]
                s = sl[0]
                ql = [h * h for h in hs]
                while len(ql) > 1:
                    ql = [ql[j] + ql[j + 1] for j in range(0, len(ql), 2)]
                q = ql[0]
                mean = jnp.sum(s) * inv_d
                ex2 = jnp.sum(q) * inv_d
                var = ex2 - mean * mean
                ve = var + EPS
                bits = lax.bitcast_convert_type(ve, jnp.int32)
                bits = jnp.int32(0x5F3759DF) - (bits >> 1)
                y = lax.bitcast_convert_type(bits, jnp.float32)
                half = jnp.float32(0.5) * ve
                for _ in range(2):
                    y = y * (jnp.float32(1.5) - half * y * y)
                yv = jnp.full((LANES,), y, dtype=jnp.float32)
                c0 = jnp.full((LANES,), -mean * y, dtype=jnp.float32)
                for t in range(KV):
                    rows_v[r, pl.ds(t * LANES, LANES)] = (
                        (hs[t] * yv + c0) * gs[t] + bs[t])

        outs = []
        for c in range(nch):
            for cp in gath[c]:
                cp.wait()
            pl.loop(c * cpw, (c + 1) * cpw, unroll=2)(pos_body)
            outs.extend(pltpu.async_copy(
                rows_v.at[pl.ds(b * spw + c * cpw, cpw)],
                out_hbm.at[b, pl.ds(sbase + c * cpw, cpw)], osem)
                for b in range(batch))
        for cp in outs:
            cp.wait()

    return k(x, embd, pos_embd, gamma, beta)


def kernel(x, embd, pos_embd, gamma, beta):
    b, s = x.shape
    return _emb_ln(x.astype(jnp.int32), embd, pos_embd, gamma, beta,
                   batch=b, seq_len=s)


# speedup vs baseline: 1.0088x; 1.0030x over previous
import functools

import jax
import jax.numpy as jnp
from jax import lax
from jax.experimental import pallas as pl
from jax.experimental.pallas import tpu as pltpu
from jax.experimental.pallas import tpu_sc as plsc

EMBD_DIM = 128
EPS = 1e-05
NC = 2
NS = 16
NW = NC * NS
LANES = 16
KV = EMBD_DIM // LANES


def _emb_ln(x, embd, pos_embd, gamma, beta, *, batch, seq_len):
    spw = seq_len // NW
    mesh = plsc.VectorSubcoreMesh(core_axis_name="c", subcore_axis_name="s")

    @functools.partial(
        pl.kernel,
        mesh=mesh,
        compiler_params=pltpu.CompilerParams(needs_layout_passes=False),
        out_type=jax.ShapeDtypeStruct((batch, seq_len, EMBD_DIM), jnp.float32),
        scratch_types=[
            pltpu.VMEM((2, 128), jnp.int32),
            pltpu.VMEM((batch * spw, EMBD_DIM), jnp.float32),
            pltpu.VMEM((spw, EMBD_DIM), jnp.float32),
            pltpu.VMEM((EMBD_DIM,), jnp.float32),
            pltpu.VMEM((EMBD_DIM,), jnp.float32),
            pltpu.SemaphoreType.DMA,
            pltpu.SemaphoreType.DMA,
            pltpu.SemaphoreType.DMA,
            pltpu.SemaphoreType.DMA,
        ],
    )
    def k(x_hbm, embd_hbm, pos_hbm, gamma_hbm, beta_hbm, out_hbm,
          idx_v, rows_v, pos_v, g_v, b_v,
          isem, msem, osem, g0sem):
        wid = lax.axis_index("s") * NC + lax.axis_index("c")
        sbase = wid * spw
        gsems = [g0sem]
        nch = len(gsems)
        cpw = spw // nch

        idx_cp = [pltpu.async_copy(
            x_hbm.at[b, pl.ds(sbase, spw)],
            idx_v.at[b // 2, pl.ds((b % 2) * spw, spw)], isem)
            for b in range(batch)]
        misc = [pltpu.async_copy(pos_hbm.at[pl.ds(sbase, spw)], pos_v, msem),
                pltpu.async_copy(gamma_hbm, g_v, msem),
                pltpu.async_copy(beta_hbm, b_v, msem)]
        for c in idx_cp:
            c.wait()
        gath = [[pltpu.async_copy(
            embd_hbm.at[idx_v.at[j]],
            rows_v.at[pl.ds(j * 128, 128)], gsems[c])
            for j in range(2)] for c in range(nch)]
        for c in misc:
            c.wait()

        gs = [g_v[pl.ds(t * LANES, LANES)] for t in range(KV)]
        bs = [b_v[pl.ds(t * LANES, LANES)] for t in range(KV)]
        inv_d = jnp.float32(1.0 / EMBD_DIM)

        def pos_body(i):
            ps = [pos_v[i, pl.ds(t * LANES, LANES)] for t in range(KV)]
            for b in range(batch):
                r = b * spw + i
                hs = [rows_v[r, pl.ds(t * LANES, LANES)] + ps[t]
                      for t in range(KV)]
                sl = hs
                while len(sl) > 1:
                    sl = [sl[j] + sl[j + 1] for j in range(0, len(sl), 2)]
                s = sl[0]
                ql = [h * h for h in hs]
                while len(ql) > 1:
                    ql = [ql[j] + ql[j + 1] for j in range(0, len(ql), 2)]
                q = ql[0]
                mean = jnp.sum(s) * inv_d
                ex2 = jnp.sum(q) * inv_d
                var = ex2 - mean * mean
                ve = var + EPS
                bits = lax.bitcast_convert_type(ve, jnp.int32)
                bits = jnp.int32(0x5F3759DF) - (bits >> 1)
                y = lax.bitcast_convert_type(bits, jnp.float32)
                half = jnp.float32(0.5) * ve
                for _ in range(2):
                    y = y * (jnp.float32(1.5) - half * y * y)
                yv = jnp.full((LANES,), y, dtype=jnp.float32)
                c0 = jnp.full((LANES,), -mean * y, dtype=jnp.float32)
                for t in range(KV):
                    rows_v[r, pl.ds(t * LANES, LANES)] = (
                        (hs[t] * yv + c0) * gs[t] + bs[t])

        outs = []
        for c in range(nch):
            for cp in gath[c]:
                cp.wait()
            pl.loop(c * cpw, (c + 1) * cpw, unroll=2)(pos_body)
            outs.extend(pltpu.async_copy(
                rows_v.at[pl.ds(b * spw + c * cpw, cpw)],
                out_hbm.at[b, pl.ds(sbase + c * cpw, cpw)], osem)
                for b in range(batch))
        for cp in outs:
            cp.wait()

    return k(x, embd, pos_embd, gamma, beta)


def kernel(x, embd, pos_embd, gamma, beta):
    b, s = x.shape
    return _emb_ln(x.astype(jnp.int32), embd, pos_embd, gamma, beta,
                   batch=b, seq_len=s)


# speedup vs baseline: 1.0152x; 1.0063x over previous
import functools

import jax
import jax.numpy as jnp
from jax import lax
from jax.experimental import pallas as pl
from jax.experimental.pallas import tpu as pltpu
from jax.experimental.pallas import tpu_sc as plsc

EMBD_DIM = 128
EPS = 1e-05
NC = 2
NS = 16
NW = NC * NS
LANES = 16
KV = EMBD_DIM // LANES


def _emb_ln(x, embd, pos_embd, gamma, beta, *, batch, seq_len):
    spw = seq_len // NW
    mesh = plsc.VectorSubcoreMesh(core_axis_name="c", subcore_axis_name="s")

    @functools.partial(
        pl.kernel,
        mesh=mesh,
        compiler_params=pltpu.CompilerParams(needs_layout_passes=False),
        out_type=jax.ShapeDtypeStruct((batch, seq_len, EMBD_DIM), jnp.float32),
        scratch_types=[
            pltpu.VMEM((2, 128), jnp.int32),
            pltpu.VMEM((batch * spw, EMBD_DIM), jnp.float32),
            pltpu.VMEM((spw, EMBD_DIM), jnp.float32),
            pltpu.VMEM((EMBD_DIM,), jnp.float32),
            pltpu.VMEM((EMBD_DIM,), jnp.float32),
            pltpu.SemaphoreType.DMA,
            pltpu.SemaphoreType.DMA,
            pltpu.SemaphoreType.DMA,
            pltpu.SemaphoreType.DMA,
        ],
    )
    def k(x_hbm, embd_hbm, pos_hbm, gamma_hbm, beta_hbm, out_hbm,
          idx_v, rows_v, pos_v, g_v, b_v,
          isem, msem, osem, g0sem):
        wid = lax.axis_index("s") * NC + lax.axis_index("c")
        sbase = wid * spw
        gsems = [g0sem]
        nch = len(gsems)
        cpw = spw // nch

        idx_cp = [pltpu.async_copy(
            x_hbm.at[b, pl.ds(sbase, spw)],
            idx_v.at[b // 2, pl.ds((b % 2) * spw, spw)], isem)
            for b in range(batch)]
        misc = [pltpu.async_copy(pos_hbm.at[pl.ds(sbase, spw)], pos_v, msem),
                pltpu.async_copy(gamma_hbm, g_v, msem),
                pltpu.async_copy(beta_hbm, b_v, msem)]
        for c in idx_cp:
            c.wait()
        gath = [[pltpu.async_copy(
            embd_hbm.at[idx_v.at[j]],
            rows_v.at[pl.ds(j * 128, 128)], gsems[c])
            for j in range(2)] for c in range(nch)]
        for c in misc:
            c.wait()

        gs = [g_v[pl.ds(t * LANES, LANES)] for t in range(KV)]
        bs = [b_v[pl.ds(t * LANES, LANES)] for t in range(KV)]
        inv_d = jnp.float32(1.0 / EMBD_DIM)

        def pos_body(i):
            ps = [pos_v[i, pl.ds(t * LANES, LANES)] for t in range(KV)]
            for b in range(batch):
                r = b * spw + i
                hs = [rows_v[r, pl.ds(t * LANES, LANES)] + ps[t]
                      for t in range(KV)]
                sl = hs
                while len(sl) > 1:
                    sl = [sl[j] + sl[j + 1] for j in range(0, len(sl), 2)]
                s = sl[0]
                ql = [h * h for h in hs]
                while len(ql) > 1:
                    ql = [ql[j] + ql[j + 1] for j in range(0, len(ql), 2)]
                q = ql[0]
                mean = jnp.sum(s) * inv_d
                ex2 = jnp.sum(q) * inv_d
                var = ex2 - mean * mean
                ve = var + EPS
                bits = lax.bitcast_convert_type(ve, jnp.int32)
                bits = jnp.int32(0x5F3759DF) - (bits >> 1)
                y = lax.bitcast_convert_type(bits, jnp.float32)
                half = jnp.float32(0.5) * ve
                for _ in range(2):
                    y = y * (jnp.float32(1.5) - half * y * y)
                yv = jnp.full((LANES,), y, dtype=jnp.float32)
                c0 = jnp.full((LANES,), -mean * y, dtype=jnp.float32)
                for t in range(KV):
                    rows_v[r, pl.ds(t * LANES, LANES)] = (
                        (hs[t] * yv + c0) * gs[t] + bs[t])

        outs = []
        for c in range(nch):
            with jax.named_scope("gwait"):
                for cp in gath[c]:
                    cp.wait()
            with jax.named_scope("ln"):
                pl.loop(c * cpw, (c + 1) * cpw, unroll=2)(pos_body)
            outs.extend(pltpu.async_copy(
                rows_v.at[pl.ds(b * spw + c * cpw, cpw)],
                out_hbm.at[b, pl.ds(sbase + c * cpw, cpw)], osem)
                for b in range(batch))
        with jax.named_scope("owait"):
            for cp in outs:
                cp.wait()

    return k(x, embd, pos_embd, gamma, beta)


def kernel(x, embd, pos_embd, gamma, beta):
    b, s = x.shape
    return _emb_ln(x.astype(jnp.int32), embd, pos_embd, gamma, beta,
                   batch=b, seq_len=s)
